# baseline (speedup 1.0000x reference)
import functools

import jax
import jax.numpy as jnp
from jax import lax
from jax.experimental import pallas as pl
from jax.experimental.pallas import tpu as pltpu
from jax.experimental.pallas import tpu_sc as plsc

VOCAB = 1000000
EMBED_DIM = 32
N_PREFIX = 20
N_CLASS = 100
BATCH = 4096
HIST = 200

B = BATCH * HIST
D = EMBED_DIM
NC, NS, L = 2, 16, 16
NW = NC * NS
IDX_COLS = 128
N_IDX_ROWS = B // IDX_COLS
ROWS_PER_W = N_IDX_ROWS // NW
CHUNK = 8
N_CHUNKS = ROWS_PER_W // CHUNK
PAD = NW


def _body(idx_hbm, tb_hbm, tn_hbm, out_hbm,
          idx_v, idxb_v, idxn_v, destb_v, destn_v, rows_b, rows_n,
          sem_g, sem_s):
    cid = lax.axis_index("c")
    sid = lax.axis_index("s")
    wid = sid * NC + cid
    dump = jnp.int32(B) + wid
    iota = lax.iota(jnp.int32, L)

    def chunk_body(g, _):
        row0 = wid * ROWS_PER_W + g * CHUNK
        pltpu.sync_copy(idx_hbm.at[pl.ds(row0, CHUNK)], idx_v)

        def compute(i, _):
            j = i // (IDX_COLS // L)
            cc = (i % (IDX_COLS // L)) * L
            vec = idx_v[j, pl.ds(cc, L)]
            is_new = vec >= VOCAB
            idxb_v[j, pl.ds(cc, L)] = jnp.minimum(vec, VOCAB - 1)
            idxn_v[j, pl.ds(cc, L)] = jnp.maximum(vec - VOCAB, 0)
            rowg = (row0 + j) * IDX_COLS + cc + iota
            destb_v[j, pl.ds(cc, L)] = jnp.where(is_new, dump, rowg)
            destn_v[j, pl.ds(cc, L)] = jnp.where(is_new, rowg, dump)
            return 0

        lax.fori_loop(0, CHUNK * (IDX_COLS // L), compute, 0)

        gathers = []
        for j in range(CHUNK):
            gathers.append(pltpu.async_copy(
                tb_hbm.at[idxb_v.at[j]], rows_b.at[j], sem_g))
            gathers.append(pltpu.async_copy(
                tn_hbm.at[idxn_v.at[j]], rows_n.at[j], sem_g))
        for h in gathers:
            h.wait()

        scatters = []
        for j in range(CHUNK):
            scatters.append(pltpu.async_copy(
                rows_b.at[j], out_hbm.at[destb_v.at[j]], sem_s))
            scatters.append(pltpu.async_copy(
                rows_n.at[j], out_hbm.at[destn_v.at[j]], sem_s))
        for h in scatters:
            h.wait()
        return 0

    lax.fori_loop(0, N_CHUNKS, chunk_body, 0)


_mesh = plsc.VectorSubcoreMesh(
    core_axis_name="c", subcore_axis_name="s", num_cores=NC, num_subcores=NS)

_emb = functools.partial(
    pl.kernel,
    out_type=jax.ShapeDtypeStruct((B + PAD, D), jnp.float32),
    mesh=_mesh,
    scratch_types=[
        pltpu.VMEM((CHUNK, IDX_COLS), jnp.int32),
        pltpu.VMEM((CHUNK, IDX_COLS), jnp.int32),
        pltpu.VMEM((CHUNK, IDX_COLS), jnp.int32),
        pltpu.VMEM((CHUNK, IDX_COLS), jnp.int32),
        pltpu.VMEM((CHUNK, IDX_COLS), jnp.int32),
        pltpu.VMEM((CHUNK, IDX_COLS, D), jnp.float32),
        pltpu.VMEM((CHUNK, IDX_COLS, D), jnp.float32),
        pltpu.SemaphoreType.DMA,
        pltpu.SemaphoreType.DMA,
    ],
    compiler_params=pltpu.CompilerParams(use_tc_tiling_on_sc=False),
)(_body)


def kernel(input, table_base, table_new):
    idx = input.astype(jnp.int32).reshape(N_IDX_ROWS, IDX_COLS)
    out = _emb(idx, table_base, table_new)
    return out[:B].reshape(BATCH, HIST, D)


# speedup vs baseline: 11.6846x; 11.6846x over previous
import functools

import jax
import jax.numpy as jnp
from jax import lax
from jax.experimental import pallas as pl
from jax.experimental.pallas import tpu as pltpu
from jax.experimental.pallas import tpu_sc as plsc

VOCAB = 1000000
EMBED_DIM = 32
N_NEW = 2000
BATCH = 4096
HIST = 200

B = BATCH * HIST
D = EMBED_DIM
NC, NS, L = 2, 16, 16
NW = NC * NS
IDX_COLS = 128
N_IDX_ROWS = B // IDX_COLS
ROWS_PER_W = N_IDX_ROWS // NW
CHUNK = 8
N_CHUNKS = ROWS_PER_W // CHUNK
GROUPS = CHUNK * (IDX_COLS // L)


def _body(idx_hbm, tb_hbm, tn_hbm, out_hbm,
          tn_v, idx_v, idxb_v, idxn_v, isnew_v, rows_v, counts_s, sem_g):
    cid = lax.axis_index("c")
    sid = lax.axis_index("s")
    wid = sid * NC + cid
    iota = lax.iota(jnp.int32, L)

    pltpu.sync_copy(tn_hbm, tn_v)

    def chunk_body(g, _):
        row0 = wid * ROWS_PER_W + g * CHUNK
        pltpu.sync_copy(idx_hbm.at[pl.ds(row0, CHUNK)], idx_v)

        def compute(i, _):
            j = i // (IDX_COLS // L)
            cc = (i % (IDX_COLS // L)) * L
            vec = idx_v[j, pl.ds(cc, L)]
            is_new = vec >= VOCAB
            rowg = (row0 + j) * IDX_COLS + cc + iota
            idxb_v[j, pl.ds(cc, L)] = jnp.where(is_new, rowg, vec)
            idxn_v[j, pl.ds(cc, L)] = jnp.clip(vec - VOCAB, 0, N_NEW - 1)
            isnew_v[j, pl.ds(cc, L)] = jnp.where(is_new, 1, 0)
            counts_s[i] = jnp.sum(jnp.where(is_new, 1, 0))
            return 0

        lax.fori_loop(0, GROUPS, compute, 0)

        gathers = [
            pltpu.async_copy(tb_hbm.at[idxb_v.at[j]],
                             rows_v.at[pl.ds(j * IDX_COLS, IDX_COLS)], sem_g)
            for j in range(CHUNK)
        ]
        for h in gathers:
            h.wait()

        def patch(i, _):
            @pl.when(counts_s[i] > 0)
            def _():
                j = i // (IDX_COLS // L)
                cc = (i % (IDX_COLS // L)) * L
                m = isnew_v[j, pl.ds(cc, L)] != 0
                idxn = idxn_v[j, pl.ds(cc, L)]
                r16 = j * IDX_COLS + cc + iota
                for col in range(D):
                    colv = jnp.full((L,), col, jnp.int32)
                    vals = plsc.load_gather(tn_v, [idxn, colv], mask=m)
                    plsc.store_scatter(rows_v, [r16, colv], vals, mask=m)
            return 0

        lax.fori_loop(0, GROUPS, patch, 0)

        pltpu.sync_copy(rows_v, out_hbm.at[pl.ds(row0 * IDX_COLS,
                                                 CHUNK * IDX_COLS)])
        return 0

    lax.fori_loop(0, N_CHUNKS, chunk_body, 0)


_mesh = plsc.VectorSubcoreMesh(
    core_axis_name="c", subcore_axis_name="s", num_cores=NC, num_subcores=NS)

_emb = functools.partial(
    pl.kernel,
    out_type=jax.ShapeDtypeStruct((B, D), jnp.float32),
    mesh=_mesh,
    scratch_types=[
        pltpu.VMEM((N_NEW, D), jnp.float32),
        pltpu.VMEM((CHUNK, IDX_COLS), jnp.int32),
        pltpu.VMEM((CHUNK, IDX_COLS), jnp.int32),
        pltpu.VMEM((CHUNK, IDX_COLS), jnp.int32),
        pltpu.VMEM((CHUNK, IDX_COLS), jnp.int32),
        pltpu.VMEM((CHUNK * IDX_COLS, D), jnp.float32),
        pltpu.SMEM((GROUPS,), jnp.int32),
        pltpu.SemaphoreType.DMA,
    ],
    compiler_params=pltpu.CompilerParams(
        use_tc_tiling_on_sc=False, needs_layout_passes=False),
)(_body)


def kernel(input, table_base, table_new):
    idx = input.astype(jnp.int32).reshape(N_IDX_ROWS, IDX_COLS)
    out = _emb(idx, table_base, table_new)
    return out.reshape(BATCH, HIST, D)


# speedup vs baseline: 11.7586x; 1.0063x over previous
import functools

import jax
import jax.numpy as jnp
from jax import lax
from jax.experimental import pallas as pl
from jax.experimental.pallas import tpu as pltpu
from jax.experimental.pallas import tpu_sc as plsc

VOCAB = 1000000
N_NEW = 2000
BATCH = 4096
HIST = 200
D = 32

NC, NS, L = 2, 16, 16
NW = NC * NS
BROWS_PER_W = BATCH // NW
CB = 8
N_CHUNKS = BROWS_PER_W // CB
CL = CB * HIST
GIDX = 100
NG_DMA = CL // GIDX
GROUPS = CL // L


def _body(idx_hbm, tb_hbm, tn_hbm, out_hbm,
          tn_v, idx_v, idxb_v, idxn_v, isnew_v, rows_v, counts_s, sem_g):
    cid = lax.axis_index("c")
    sid = lax.axis_index("s")
    wid = sid * NC + cid
    iota = lax.iota(jnp.int32, L)

    pltpu.sync_copy(tn_hbm, tn_v)

    def chunk_body(k, _):
        b0 = wid * BROWS_PER_W + k * CB
        pltpu.sync_copy(idx_hbm.at[pl.ds(b0, CB)], idx_v)

        def compute(g, _):
            o16 = g * L + iota
            vec = plsc.load_gather(idx_v, [o16 // HIST, o16 % HIST])
            is_new = vec >= VOCAB
            flatrow = b0 * HIST + o16
            bi, bj = o16 // GIDX, o16 % GIDX
            plsc.store_scatter(idxb_v, [bi, bj],
                               jnp.where(is_new, flatrow, vec))
            plsc.store_scatter(idxn_v, [bi, bj],
                               jnp.clip(vec - VOCAB, 0, N_NEW - 1))
            plsc.store_scatter(isnew_v, [bi, bj], jnp.where(is_new, 1, 0))
            counts_s[g] = jnp.sum(jnp.where(is_new, 1, 0))
            return 0

        lax.fori_loop(0, GROUPS, compute, 0)

        gathers = [
            pltpu.async_copy(tb_hbm.at[idxb_v.at[j]],
                             rows_v.at[pl.ds(j * GIDX, GIDX)], sem_g)
            for j in range(NG_DMA)
        ]
        for h in gathers:
            h.wait()

        def patch(g, _):
            @pl.when(counts_s[g] > 0)
            def _():
                o16 = g * L + iota
                bi, bj = o16 // GIDX, o16 % GIDX
                m = plsc.load_gather(isnew_v, [bi, bj]) != 0
                idxn = plsc.load_gather(idxn_v, [bi, bj])
                for col in range(D):
                    colv = jnp.full((L,), col, jnp.int32)
                    vals = plsc.load_gather(tn_v, [idxn, colv], mask=m)
                    plsc.store_scatter(rows_v, [o16, colv], vals, mask=m)
            return 0

        lax.fori_loop(0, GROUPS, patch, 0)

        outs = [
            pltpu.async_copy(rows_v.at[pl.ds(r * HIST, HIST)],
                             out_hbm.at[b0 + r], sem_g)
            for r in range(CB)
        ]
        for h in outs:
            h.wait()
        return 0

    lax.fori_loop(0, N_CHUNKS, chunk_body, 0)


_mesh = plsc.VectorSubcoreMesh(
    core_axis_name="c", subcore_axis_name="s", num_cores=NC, num_subcores=NS)

_emb = functools.partial(
    pl.kernel,
    out_type=jax.ShapeDtypeStruct((BATCH, HIST, D), jnp.float32),
    mesh=_mesh,
    scratch_types=[
        pltpu.VMEM((N_NEW, D), jnp.float32),
        pltpu.VMEM((CB, HIST), jnp.int32),
        pltpu.VMEM((NG_DMA, GIDX), jnp.int32),
        pltpu.VMEM((NG_DMA, GIDX), jnp.int32),
        pltpu.VMEM((NG_DMA, GIDX), jnp.int32),
        pltpu.VMEM((CL, D), jnp.float32),
        pltpu.SMEM((GROUPS,), jnp.int32),
        pltpu.SemaphoreType.DMA,
    ],
    compiler_params=pltpu.CompilerParams(
        use_tc_tiling_on_sc=False, needs_layout_passes=False),
)(_body)


def kernel(input, table_base, table_new):
    return _emb(input.astype(jnp.int32), table_base, table_new)
